# initial kernel scaffold (unmeasured)
import jax
import jax.numpy as jnp
from jax import lax
from jax.experimental import pallas as pl
from jax.experimental.pallas import tpu as pltpu


def kernel(
    x,
):
    def body(*refs):
        pass

    out_shape = jax.ShapeDtypeStruct(..., jnp.float32)
    return pl.pallas_call(body, out_shape=out_shape)(...)


# baseline (device time: 155093 ns/iter reference)
import functools

import jax
import jax.numpy as jnp
from jax import lax
from jax.experimental import pallas as pl
from jax.experimental.pallas import tpu as pltpu

N_Z = 4
M = 2048
N_COL = 2048
CHUNK = N_COL // N_Z


def kernel(x):
    def body(x_ref, out_ref, comm_ref, send_sems, recv_sems):
        my_x = lax.axis_index("x")
        my_y = lax.axis_index("y")
        my_z = lax.axis_index("z")
        left = lax.rem(my_z + (N_Z - 1), N_Z)
        right = lax.rem(my_z + 1, N_Z)

        barrier_sem = pltpu.get_barrier_semaphore()
        for nbr in (left, right):
            pl.semaphore_signal(
                barrier_sem,
                inc=1,
                device_id=(my_x, my_y, nbr),
                device_id_type=pl.DeviceIdType.MESH,
            )
        pl.semaphore_wait(barrier_sem, 2)

        def chunk(idx):
            return x_ref[0, :, pl.ds(idx * CHUNK, CHUNK)]

        comm_ref[N_Z - 1, :, :] = chunk(lax.rem(my_z + (N_Z - 1), N_Z))

        for s in range(N_Z - 1):
            src_slot = (N_Z - 1) if s == 0 else s - 1
            rdma = pltpu.make_async_remote_copy(
                src_ref=comm_ref.at[src_slot],
                dst_ref=comm_ref.at[s],
                send_sem=send_sems.at[s],
                recv_sem=recv_sems.at[s],
                device_id=(my_x, my_y, right),
                device_id_type=pl.DeviceIdType.MESH,
            )
            rdma.start()
            rdma.wait()

            idx = lax.rem(my_z + (N_Z - 2 - s) + N_Z, N_Z)
            if s < N_Z - 2:
                comm_ref[s, :, :] = comm_ref[s, :, :] + chunk(idx)
            else:
                out_ref[:, :] = comm_ref[s, :, :] + chunk(idx)

        @functools.partial(
            pl.run_scoped, second_barrier=pltpu.SemaphoreType.REGULAR
        )
        def _(second_barrier):
            for nbr in (left, right):
                pl.semaphore_signal(
                    second_barrier,
                    inc=1,
                    device_id=(my_x, my_y, nbr),
                    device_id_type=pl.DeviceIdType.MESH,
                )
            pl.semaphore_wait(second_barrier, 2)

    return pl.pallas_call(
        body,
        out_shape=jax.ShapeDtypeStruct((M, CHUNK), jnp.float32),
        in_specs=[pl.BlockSpec(memory_space=pltpu.VMEM)],
        out_specs=pl.BlockSpec(memory_space=pltpu.VMEM),
        scratch_shapes=[
            pltpu.VMEM((N_Z, M, CHUNK), jnp.float32),
            pltpu.SemaphoreType.DMA((N_Z - 1,)),
            pltpu.SemaphoreType.DMA((N_Z - 1,)),
        ],
        compiler_params=pltpu.CompilerParams(collective_id=0),
    )(x)


# device time: 70706 ns/iter; 2.1935x vs baseline; 2.1935x over previous
import jax
import jax.numpy as jnp
from jax import lax
from jax.experimental import pallas as pl
from jax.experimental.pallas import tpu as pltpu

N_Z = 4
N_G = 4
M = 2048
N_COL = 2048
CHUNK = N_COL // N_Z
BAND = M // N_G


def kernel(x):
    def body(x_ref, out_ref, p1_buf, res_buf, sz_send, sz_recv, sxy_send, sxy_recv):
        my_x = lax.axis_index("x")
        my_y = lax.axis_index("y")
        my_z = lax.axis_index("z")
        p = 2 * my_x + my_y
        r0 = p * BAND

        peers = []
        for d in (1, 2, 3):
            peers.append((my_x, my_y, lax.rem(my_z + d, N_Z)))
        peers.append((1 - my_x, my_y, my_z))
        peers.append((my_x, 1 - my_y, my_z))
        peers.append((1 - my_x, 1 - my_y, my_z))

        barrier_sem = pltpu.get_barrier_semaphore()
        for dev in peers:
            pl.semaphore_signal(
                barrier_sem, inc=1, device_id=dev,
                device_id_type=pl.DeviceIdType.MESH,
            )
        pl.semaphore_wait(barrier_sem, len(peers))

        z_rdmas = []
        for d in (1, 2, 3):
            tz = lax.rem(my_z + d, N_Z)
            rdma = pltpu.make_async_remote_copy(
                src_ref=x_ref.at[0, pl.ds(r0, BAND), pl.ds(tz * CHUNK, CHUNK)],
                dst_ref=p1_buf.at[d - 1],
                send_sem=sz_send.at[d - 1],
                recv_sem=sz_recv.at[d - 1],
                device_id=(my_x, my_y, tz),
                device_id_type=pl.DeviceIdType.MESH,
            )
            rdma.start()
            z_rdmas.append(rdma)

        for s in (0, 1, 2):
            pltpu.make_async_remote_copy(
                src_ref=p1_buf.at[s],
                dst_ref=p1_buf.at[s],
                send_sem=sz_send.at[s],
                recv_sem=sz_recv.at[s],
                device_id=(my_x, my_y, my_z),
                device_id_type=pl.DeviceIdType.MESH,
            ).wait_recv()

        own = x_ref[0, pl.ds(r0, BAND), pl.ds(my_z * CHUNK, CHUNK)]
        res = own + p1_buf[0] + p1_buf[1] + p1_buf[2]
        res_buf[:, :] = res
        out_ref[pl.ds(r0, BAND), :] = res

        xy_rdmas = []
        for d in (1, 2, 3):
            q = lax.rem(p + d, N_G)
            tx = lax.div(q, 2)
            ty = lax.rem(q, 2)
            rdma = pltpu.make_async_remote_copy(
                src_ref=res_buf,
                dst_ref=out_ref.at[pl.ds(r0, BAND), :],
                send_sem=sxy_send.at[d - 1],
                recv_sem=sxy_recv.at[d - 1],
                device_id=(tx, ty, my_z),
                device_id_type=pl.DeviceIdType.MESH,
            )
            rdma.start()
            xy_rdmas.append(rdma)

        for s in (0, 1, 2):
            pltpu.make_async_remote_copy(
                src_ref=res_buf,
                dst_ref=res_buf,
                send_sem=sxy_send.at[s],
                recv_sem=sxy_recv.at[s],
                device_id=(my_x, my_y, my_z),
                device_id_type=pl.DeviceIdType.MESH,
            ).wait_recv()

        for rdma in z_rdmas + xy_rdmas:
            rdma.wait_send()

    return pl.pallas_call(
        body,
        out_shape=jax.ShapeDtypeStruct((M, CHUNK), jnp.float32),
        in_specs=[pl.BlockSpec(memory_space=pltpu.VMEM)],
        out_specs=pl.BlockSpec(memory_space=pltpu.VMEM),
        scratch_shapes=[
            pltpu.VMEM((3, BAND, CHUNK), jnp.float32),
            pltpu.VMEM((BAND, CHUNK), jnp.float32),
            pltpu.SemaphoreType.DMA((3,)),
            pltpu.SemaphoreType.DMA((3,)),
            pltpu.SemaphoreType.DMA((3,)),
            pltpu.SemaphoreType.DMA((3,)),
        ],
        compiler_params=pltpu.CompilerParams(collective_id=0),
    )(x)


# device time: 66535 ns/iter; 2.3310x vs baseline; 1.0627x over previous
import jax
import jax.numpy as jnp
from jax import lax
from jax.experimental import pallas as pl
from jax.experimental.pallas import tpu as pltpu

N_Z = 4
N_G = 4
M = 2048
N_COL = 2048
CHUNK = N_COL // N_Z
BAND = M // N_G
HALF = CHUNK // 2


def kernel(x):
    def body(x_ref, out_ref, p1_buf, res_buf, sz_send, sz_recv, sxy_send, sxy_recv):
        my_x = lax.axis_index("x")
        my_y = lax.axis_index("y")
        my_z = lax.axis_index("z")
        p = 2 * my_x + my_y
        r0 = p * BAND

        peers = [(my_x, my_y, lax.rem(my_z + d, N_Z)) for d in (1, 2, 3)]
        peers += [
            (1 - my_x, my_y, my_z),
            (my_x, 1 - my_y, my_z),
            (1 - my_x, 1 - my_y, my_z),
        ]

        barrier_sem = pltpu.get_barrier_semaphore()
        for dev in peers:
            pl.semaphore_signal(
                barrier_sem, inc=1, device_id=dev,
                device_id_type=pl.DeviceIdType.MESH,
            )
        pl.semaphore_wait(barrier_sem, len(peers))

        pending = []
        for h in (0, 1):
            for d in (1, 2, 3):
                tz = lax.rem(my_z + d, N_Z)
                slot = (d - 1) * 2 + h
                rdma = pltpu.make_async_remote_copy(
                    src_ref=x_ref.at[
                        0, pl.ds(r0, BAND), pl.ds(tz * CHUNK + h * HALF, HALF)
                    ],
                    dst_ref=p1_buf.at[slot],
                    send_sem=sz_send.at[slot],
                    recv_sem=sz_recv.at[slot],
                    device_id=(my_x, my_y, tz),
                    device_id_type=pl.DeviceIdType.MESH,
                )
                rdma.start()
                pending.append(rdma)

        xy_targets = []
        for d in (1, 2, 3):
            q = lax.rem(p + d, N_G)
            xy_targets.append((lax.div(q, 2), lax.rem(q, 2)))

        for h in (0, 1):
            for d in (1, 2, 3):
                slot = (d - 1) * 2 + h
                pltpu.make_async_remote_copy(
                    src_ref=p1_buf.at[slot],
                    dst_ref=p1_buf.at[slot],
                    send_sem=sz_send.at[slot],
                    recv_sem=sz_recv.at[slot],
                    device_id=(my_x, my_y, my_z),
                    device_id_type=pl.DeviceIdType.MESH,
                ).wait_recv()

            own = x_ref[0, pl.ds(r0, BAND), pl.ds(my_z * CHUNK + h * HALF, HALF)]
            res = own + p1_buf[0 + h] + p1_buf[2 + h] + p1_buf[4 + h]
            res_buf[:, pl.ds(h * HALF, HALF)] = res
            out_ref[pl.ds(r0, BAND), pl.ds(h * HALF, HALF)] = res

            for d in (1, 2, 3):
                slot = (d - 1) * 2 + h
                tx, ty = xy_targets[d - 1]
                rdma = pltpu.make_async_remote_copy(
                    src_ref=res_buf.at[:, pl.ds(h * HALF, HALF)],
                    dst_ref=out_ref.at[pl.ds(r0, BAND), pl.ds(h * HALF, HALF)],
                    send_sem=sxy_send.at[slot],
                    recv_sem=sxy_recv.at[slot],
                    device_id=(tx, ty, my_z),
                    device_id_type=pl.DeviceIdType.MESH,
                )
                rdma.start()
                pending.append(rdma)

        for h in (0, 1):
            for d in (1, 2, 3):
                slot = (d - 1) * 2 + h
                pltpu.make_async_remote_copy(
                    src_ref=res_buf.at[:, pl.ds(h * HALF, HALF)],
                    dst_ref=res_buf.at[:, pl.ds(h * HALF, HALF)],
                    send_sem=sxy_send.at[slot],
                    recv_sem=sxy_recv.at[slot],
                    device_id=(my_x, my_y, my_z),
                    device_id_type=pl.DeviceIdType.MESH,
                ).wait_recv()

        for rdma in pending:
            rdma.wait_send()

    return pl.pallas_call(
        body,
        out_shape=jax.ShapeDtypeStruct((M, CHUNK), jnp.float32),
        in_specs=[pl.BlockSpec(memory_space=pltpu.VMEM)],
        out_specs=pl.BlockSpec(memory_space=pltpu.VMEM),
        scratch_shapes=[
            pltpu.VMEM((6, BAND, HALF), jnp.float32),
            pltpu.VMEM((BAND, CHUNK), jnp.float32),
            pltpu.SemaphoreType.DMA((6,)),
            pltpu.SemaphoreType.DMA((6,)),
            pltpu.SemaphoreType.DMA((6,)),
            pltpu.SemaphoreType.DMA((6,)),
        ],
        compiler_params=pltpu.CompilerParams(collective_id=0),
    )(x)
